# baseline (speedup 1.0000x reference)
import functools

import jax
import jax.numpy as jnp
from jax.experimental import pallas as pl

_B = 1024
_D = 32
_R = 8
_NBLK = _B // _R
_BIG = 1 << 30


def _tc_kernel(emb_ref, embT_ref, embblk_ref, labr_ref, labc_ref,
               diff_ref, pos_ref, neg_ref):
    i = pl.program_id(0)
    emb = emb_ref[...]
    emb_blk = embblk_ref[...]
    diff_ref[...] = emb[None, :, :] - emb_blk[:, None, :]

    embT = embT_ref[...]
    dot = jnp.dot(emb_blk, embT, preferred_element_type=jnp.float32,
                  precision=jax.lax.Precision.HIGHEST)
    nj = jnp.sum(embT * embT, axis=0, keepdims=True)
    ni = jnp.sum(emb_blk * emb_blk, axis=1, keepdims=True)
    d2 = ni + nj - 2.0 * dot

    labr = labr_ref[...]
    labc = labc_ref[...]
    same = labr == labc
    col = jax.lax.broadcasted_iota(jnp.int32, (_R, _B), 1)
    row = jax.lax.broadcasted_iota(jnp.int32, (_R, _B), 0) + i * _R
    noteye = col != row

    pos_d = jnp.where(same & noteye, d2, -jnp.inf)
    neg_d = jnp.where(~same, d2, jnp.inf)

    pmax = jnp.max(pos_d, axis=1, keepdims=True)
    pidx = jnp.min(jnp.where(pos_d == pmax, col, _BIG), axis=1)
    nmin = jnp.min(neg_d, axis=1, keepdims=True)
    nidx = jnp.min(jnp.where(neg_d == nmin, col, _BIG), axis=1)

    pos_ref[...] = pidx.reshape(1, 1, _R)
    neg_ref[...] = nidx.reshape(1, 1, _R)


@jax.jit
def kernel(embeddings, labels):
    embT = embeddings.T
    labr = labels.reshape(_B, 1)
    labc = labels.reshape(1, _B)
    diff, pos3, neg3 = pl.pallas_call(
        _tc_kernel,
        grid=(_NBLK,),
        in_specs=[
            pl.BlockSpec((_B, _D), lambda i: (0, 0)),
            pl.BlockSpec((_D, _B), lambda i: (0, 0)),
            pl.BlockSpec((_R, _D), lambda i: (i, 0)),
            pl.BlockSpec((_R, 1), lambda i: (i, 0)),
            pl.BlockSpec((1, _B), lambda i: (0, 0)),
        ],
        out_specs=[
            pl.BlockSpec((_R, _B, _D), lambda i: (i, 0, 0)),
            pl.BlockSpec((1, 1, _R), lambda i: (i, 0, 0)),
            pl.BlockSpec((1, 1, _R), lambda i: (i, 0, 0)),
        ],
        out_shape=[
            jax.ShapeDtypeStruct((_B, _B, _D), jnp.float32),
            jax.ShapeDtypeStruct((_NBLK, 1, _R), jnp.int32),
            jax.ShapeDtypeStruct((_NBLK, 1, _R), jnp.int32),
        ],
    )(embeddings, embT, embeddings, labr, labc)
    return diff, pos3.reshape(_B), neg3.reshape(_B)


# speedup vs baseline: 1.4784x; 1.4784x over previous
import jax
import jax.numpy as jnp
from jax.experimental import pallas as pl

_B = 1024
_D = 32
_R = 8
_NBLK = _B // _R
_C = _B * _D // 128
_BIG = 1 << 30


def _tc_kernel(embv_ref, embrep_ref, embT_ref, embblk_ref, labr_ref, labc_ref,
               diff_ref, pos_ref, neg_ref):
    i = pl.program_id(0)
    embv = embv_ref[...]
    embrep = embrep_ref[...]
    diff_ref[...] = embv - embrep

    emb_blk = embblk_ref[...]
    embT = embT_ref[...]
    dot = jnp.dot(emb_blk, embT, preferred_element_type=jnp.float32,
                  precision=jax.lax.Precision.HIGHEST)
    nj = jnp.sum(embT * embT, axis=0, keepdims=True)
    ni = jnp.sum(emb_blk * emb_blk, axis=1, keepdims=True)
    d2 = ni + nj - 2.0 * dot

    labr = labr_ref[...]
    labc = labc_ref[...]
    same = labr == labc
    col = jax.lax.broadcasted_iota(jnp.int32, (_R, _B), 1)
    row = jax.lax.broadcasted_iota(jnp.int32, (_R, _B), 0) + i * _R
    noteye = col != row

    pos_d = jnp.where(same & noteye, d2, -jnp.inf)
    neg_d = jnp.where(~same, d2, jnp.inf)

    pmax = jnp.max(pos_d, axis=1, keepdims=True)
    pidx = jnp.min(jnp.where(pos_d == pmax, col, _BIG), axis=1)
    nmin = jnp.min(neg_d, axis=1, keepdims=True)
    nidx = jnp.min(jnp.where(neg_d == nmin, col, _BIG), axis=1)

    pos_ref[...] = pidx.reshape(1, 1, _R)
    neg_ref[...] = nidx.reshape(1, 1, _R)


@jax.jit
def kernel(embeddings, labels):
    embv = embeddings.reshape(1, _C, 128)
    embrep = jnp.tile(embeddings, (1, 128 // _D)).reshape(_B, 1, 128)
    embT = embeddings.T
    labr = labels.reshape(_B, 1)
    labc = labels.reshape(1, _B)
    diff, pos3, neg3 = pl.pallas_call(
        _tc_kernel,
        grid=(_NBLK,),
        in_specs=[
            pl.BlockSpec((1, _C, 128), lambda i: (0, 0, 0)),
            pl.BlockSpec((_R, 1, 128), lambda i: (i, 0, 0)),
            pl.BlockSpec((_D, _B), lambda i: (0, 0)),
            pl.BlockSpec((_R, _D), lambda i: (i, 0)),
            pl.BlockSpec((_R, 1), lambda i: (i, 0)),
            pl.BlockSpec((1, _B), lambda i: (0, 0)),
        ],
        out_specs=[
            pl.BlockSpec((_R, _C, 128), lambda i: (i, 0, 0)),
            pl.BlockSpec((1, 1, _R), lambda i: (i, 0, 0)),
            pl.BlockSpec((1, 1, _R), lambda i: (i, 0, 0)),
        ],
        out_shape=[
            jax.ShapeDtypeStruct((_B, _C, 128), jnp.float32),
            jax.ShapeDtypeStruct((_NBLK, 1, _R), jnp.int32),
            jax.ShapeDtypeStruct((_NBLK, 1, _R), jnp.int32),
        ],
    )(embv, embrep, embT, embeddings, labr, labc)
    return (diff.reshape(_B, _B, _D), pos3.reshape(_B), neg3.reshape(_B))


# speedup vs baseline: 9.2096x; 6.2295x over previous
import jax
import jax.numpy as jnp
from jax.experimental import pallas as pl

_B = 1024
_D = 32
_R = 32
_NBLK = _B // _R
_BIG = 1 << 30


def _tc_kernel(embT_ref, embblk_ref, labr_ref, labc_ref,
               diff_ref, pos_ref, neg_ref):
    i = pl.program_id(0)
    embT = embT_ref[...]
    emb_blk = embblk_ref[...]
    diff_ref[...] = embT[None, :, :] - emb_blk[:, :, None]

    dot = jnp.dot(emb_blk, embT, preferred_element_type=jnp.float32,
                  precision=jax.lax.Precision.HIGHEST)
    nj = jnp.sum(embT * embT, axis=0, keepdims=True)
    ni = jnp.sum(emb_blk * emb_blk, axis=1, keepdims=True)
    d2 = ni + nj - 2.0 * dot

    labr = labr_ref[...]
    labc = labc_ref[...]
    same = labr == labc
    col = jax.lax.broadcasted_iota(jnp.int32, (_R, _B), 1)
    row = jax.lax.broadcasted_iota(jnp.int32, (_R, _B), 0) + i * _R
    noteye = col != row

    pos_d = jnp.where(same & noteye, d2, -jnp.inf)
    neg_d = jnp.where(~same, d2, jnp.inf)

    pmax = jnp.max(pos_d, axis=1, keepdims=True)
    pidx = jnp.min(jnp.where(pos_d == pmax, col, _BIG), axis=1)
    nmin = jnp.min(neg_d, axis=1, keepdims=True)
    nidx = jnp.min(jnp.where(neg_d == nmin, col, _BIG), axis=1)

    pos_ref[...] = pidx.reshape(1, 1, _R)
    neg_ref[...] = nidx.reshape(1, 1, _R)


@jax.jit
def kernel(embeddings, labels):
    embT = embeddings.T
    labr = labels.reshape(_B, 1)
    labc = labels.reshape(1, _B)
    diff3, pos3, neg3 = pl.pallas_call(
        _tc_kernel,
        grid=(_NBLK,),
        in_specs=[
            pl.BlockSpec((_D, _B), lambda i: (0, 0)),
            pl.BlockSpec((_R, _D), lambda i: (i, 0)),
            pl.BlockSpec((_R, 1), lambda i: (i, 0)),
            pl.BlockSpec((1, _B), lambda i: (0, 0)),
        ],
        out_specs=[
            pl.BlockSpec((_R, _D, _B), lambda i: (i, 0, 0)),
            pl.BlockSpec((1, 1, _R), lambda i: (i, 0, 0)),
            pl.BlockSpec((1, 1, _R), lambda i: (i, 0, 0)),
        ],
        out_shape=[
            jax.ShapeDtypeStruct((_B, _D, _B), jnp.float32),
            jax.ShapeDtypeStruct((_NBLK, 1, _R), jnp.int32),
            jax.ShapeDtypeStruct((_NBLK, 1, _R), jnp.int32),
        ],
    )(embT, embeddings, labr, labc)
    return (jnp.transpose(diff3, (0, 2, 1)), pos3.reshape(_B), neg3.reshape(_B))
